# baseline (speedup 1.0000x reference)
import functools

import jax
import jax.numpy as jnp
from jax import lax
from jax.experimental import pallas as pl
from jax.experimental.pallas import tpu as pltpu
from jax.experimental.pallas import tpu_sc as plsc

_B, _F = 4096, 26
_DW, _DD = 8, 16
_DIN = _F * _DD
_NC, _NS = 2, 16
_NW = _NC * _NS
_BPW = _B // _NW
_IPW = _BPW * _F
_G = 128
_NG = _IPW // _G
_BH = _BPW // 2


def _sc_body(xf_hbm, seg_hbm, zero_hbm, wide_hbm, deep_hbm, deep_out, wide_out,
             idx_v, seg_v, dtmp, wtmp, buf, spacc, sem_d, sem_w):
    sid = lax.axis_index("s")
    wid = sid * _NC + lax.axis_index("c")
    b0 = wid * _BPW
    off = sid * _BPW
    pltpu.sync_copy(xf_hbm.at[pl.ds(b0 * _F, _IPW)], idx_v)
    pltpu.sync_copy(seg_hbm.at[sid], seg_v)
    pltpu.sync_copy(zero_hbm, spacc.at[pl.ds(off, _BPW)])
    dc = [pltpu.async_copy(deep_hbm.at[idx_v.at[pl.ds(g * _G, _G)]],
                           dtmp.at[pl.ds(g * _G, _G)], sem_d)
          for g in range(_NG)]
    wc = [pltpu.async_copy(wide_hbm.at[idx_v.at[pl.ds(g * _G, _G)]],
                           wtmp.at[pl.ds(g * _G, _G)], sem_w)
          for g in range(_NG)]
    for c in wc:
        c.wait()
    pltpu.sync_copy(wtmp, spacc.at[seg_v], add=True)
    pltpu.sync_copy(spacc.at[pl.ds(off, _BPW)], wide_out.at[pl.ds(b0, _BPW)])
    for c in dc:
        c.wait()
    for h in range(2):
        def relayout(b, _):
            base = (h * _BH + b) * _F
            for f in range(_F):
                buf[b, pl.ds(f * _DD, _DD)] = dtmp[base + f, :]
            return 0
        lax.fori_loop(0, _BH, relayout, 0)
        pltpu.sync_copy(buf, deep_out.at[pl.ds(b0 + h * _BH, _BH)])


_sc_gather = functools.partial(
    pl.kernel,
    out_type=(
        jax.ShapeDtypeStruct((_B, _DIN), jnp.float32),
        jax.ShapeDtypeStruct((_B, _DW), jnp.float32),
    ),
    mesh=plsc.VectorSubcoreMesh(
        core_axis_name="c", subcore_axis_name="s", num_cores=_NC, num_subcores=_NS
    ),
    compiler_params=pltpu.CompilerParams(use_tc_tiling_on_sc=False),
    scratch_types=[
        pltpu.VMEM((_IPW,), jnp.int32),
        pltpu.VMEM((_IPW,), jnp.int32),
        pltpu.VMEM((_IPW, _DD), jnp.float32),
        pltpu.VMEM((_IPW, _DW), jnp.float32),
        pltpu.VMEM((_BH, _DIN), jnp.float32),
        pltpu.VMEM_SHARED((_NS * _BPW, _DW), jnp.float32),
        pltpu.SemaphoreType.DMA,
        pltpu.SemaphoreType.DMA,
    ],
)(_sc_body)


_BLK = 512


def _tc_body(deep_ref, wide_ref, g_ref, bta_ref,
             w1, b1, w2, b2, w3, b3, w4, b4, out_ref):
    wide_sum = jnp.sum(wide_ref[...], axis=1, keepdims=True)
    h0 = deep_ref[...]
    mu = jnp.mean(h0, axis=1, keepdims=True)
    xc = h0 - mu
    var = jnp.mean(xc * xc, axis=1, keepdims=True)
    h = xc * lax.rsqrt(var + 1e-5) * g_ref[...] + bta_ref[...]
    h = jnp.maximum(jnp.dot(h, w1[...], preferred_element_type=jnp.float32) + b1[...], 0.0)
    h = jnp.maximum(jnp.dot(h, w2[...], preferred_element_type=jnp.float32) + b2[...], 0.0)
    h = jnp.maximum(jnp.dot(h, w3[...], preferred_element_type=jnp.float32) + b3[...], 0.0)
    z = jnp.dot(h, w4[...], preferred_element_type=jnp.float32) + b4[...] + wide_sum
    out_ref[...] = jax.nn.sigmoid(z)


def _full(shape):
    return pl.BlockSpec(shape, lambda i: (0, 0))


_tc_mlp = pl.pallas_call(
    _tc_body,
    grid=(_B // _BLK,),
    in_specs=[
        pl.BlockSpec((_BLK, _DIN), lambda i: (i, 0)),
        pl.BlockSpec((_BLK, _DW), lambda i: (i, 0)),
        _full((1, _DIN)),
        _full((1, _DIN)),
        _full((_DIN, 512)),
        _full((1, 512)),
        _full((512, 256)),
        _full((1, 256)),
        _full((256, 128)),
        _full((1, 128)),
        _full((128, 1)),
        _full((1, 1)),
    ],
    out_specs=pl.BlockSpec((_BLK, 1), lambda i: (i, 0)),
    out_shape=jax.ShapeDtypeStruct((_B, 1), jnp.float32),
)


def kernel(x, wide_table, deep_table, ln_gamma, ln_beta,
           W1, b1, W2, b2, W3, b3, W4, b4):
    xf = x.reshape(_B * _F)
    seg = (jnp.arange(_IPW, dtype=jnp.int32) // _F)[None, :] \
        + _BPW * jnp.arange(_NS, dtype=jnp.int32)[:, None]
    zero = jnp.zeros((_BPW, _DW), jnp.float32)
    deep_cat, wide8 = _sc_gather(xf, seg, zero, wide_table, deep_table)
    return _tc_mlp(
        deep_cat, wide8,
        ln_gamma.reshape(1, _DIN), ln_beta.reshape(1, _DIN),
        W1, b1.reshape(1, 512), W2, b2.reshape(1, 256),
        W3, b3.reshape(1, 128), W4, b4.reshape(1, 1),
    )
